# Initial kernel scaffold; baseline (speedup 1.0000x reference)
import jax
import jax.numpy as jnp
from jax.experimental import pallas as pl


def kernel(X, features, emb):
    raise NotImplementedError("write your pallas kernel here")


# speedup vs baseline: 2.4349x; 2.4349x over previous
import dataclasses

import jax
import jax.numpy as jnp
from jax import lax
from jax.experimental import pallas as pl
from jax.experimental.pallas import tpu as pltpu
from jax.experimental.pallas import tpu_sc as plsc

B, T, N = 16, 2048, 16
H = 64
NF = 7
NAA = 21
HE = H - NF

NC, NS, L = 2, 16, 16
TCH = 256
NCHUNK = T // (NC * TCH)
HG = 8
NTV = TCH // L


def _sc_kernel(x_hbm, f_hbm, e_hbm, out_hbm, xv, fv, ev, ov):
    b = lax.axis_index("s")
    th = lax.axis_index("c")

    pltpu.sync_copy(e_hbm, ev)

    i16 = lax.iota(jnp.int32, L)
    ix16 = i16 * N
    ix112 = i16 * (N * NF)

    @pl.loop(0, NCHUNK)
    def _chunk(ck):
        t0 = pl.multiple_of(th * (NCHUNK * TCH) + ck * TCH, TCH)
        pltpu.sync_copy(x_hbm.at[b, pl.ds(t0 * N, TCH * N)], xv)
        pltpu.sync_copy(f_hbm.at[b, pl.ds(t0 * N * NF, TCH * N * NF)], fv)

        for hg in range(H // HG):
            @pl.loop(0, N)
            def _n(n):
                @pl.loop(0, NTV)
                def _tv(tv):
                    aa = plsc.load_gather(xv, [ix16 + (tv * (L * N) + n)])
                    toff = tv * L
                    for hl in range(HG):
                        h = hg * HG + hl
                        if h < HE:
                            val = plsc.load_gather(ev, [aa * HE + h])
                        else:
                            f = h - HE
                            val = plsc.load_gather(
                                fv, [ix112 + (tv * (L * N * NF) + n * NF + f)])
                        ov[hl, n, pl.ds(toff, L)] = val

            pltpu.sync_copy(
                ov, out_hbm.at[b, pl.ds(hg * HG, HG), :, pl.ds(t0, TCH)])


def kernel(X, features, emb):
    x2 = X.astype(jnp.int32).reshape(B, T * N)
    f2 = features.reshape(B, T * N * NF)
    epad = jnp.zeros((NAA * HE + 19,), jnp.float32)
    epad = lax.dynamic_update_slice(epad, emb.reshape(-1), (0,))

    cp = pltpu.CompilerParams()
    if "needs_layout_passes" in pltpu.CompilerParams.__dataclass_fields__:
        cp = dataclasses.replace(cp, needs_layout_passes=False)
    mesh = plsc.VectorSubcoreMesh(core_axis_name="c", subcore_axis_name="s")
    k = pl.kernel(
        _sc_kernel,
        out_type=jax.ShapeDtypeStruct((B, H, N, T), jnp.float32),
        mesh=mesh,
        compiler_params=cp,
        scratch_types=[
            pltpu.VMEM((TCH * N,), jnp.int32),
            pltpu.VMEM((TCH * N * NF,), jnp.float32),
            pltpu.VMEM((NAA * HE + 19,), jnp.float32),
            pltpu.VMEM((HG, N, TCH), jnp.float32),
        ],
    )
    return k(x2, f2, epad)


# speedup vs baseline: 3.6745x; 1.5091x over previous
import dataclasses

import jax
import jax.numpy as jnp
from jax import lax
from jax.experimental import pallas as pl
from jax.experimental.pallas import tpu as pltpu
from jax.experimental.pallas import tpu_sc as plsc

B, T, N = 16, 2048, 16
H = 64
NF = 7
NAA = 21
HE = H - NF

NC, NS, L = 2, 16, 16
TCH = 256
NCHUNK = T // (NC * TCH)
HG = 8
NTV = TCH // L


def _sc_kernel(x_hbm, f_hbm, e_hbm, out_hbm, xv, fv, ev, ov):
    b = lax.axis_index("s")
    th = lax.axis_index("c")

    pltpu.sync_copy(e_hbm, ev)

    i16 = lax.iota(jnp.int32, L)
    ix16 = i16 * N
    ix112 = i16 * (N * NF)

    @pl.loop(0, NCHUNK)
    def _chunk(ck):
        t0 = pl.multiple_of(th * (NCHUNK * TCH) + ck * TCH, TCH)
        pltpu.sync_copy(x_hbm.at[b, pl.ds(t0 * N, TCH * N)], xv)
        pltpu.sync_copy(f_hbm.at[b, pl.ds(t0 * N * NF, TCH * N * NF)], fv)

        for hg in range(H // HG):
            @pl.loop(0, N)
            def _n(n):
                @plsc.parallel_loop(0, NTV, unroll=4)
                def _tv(tv):
                    aa = plsc.load_gather(xv, [ix16 + (tv * (L * N) + n)])
                    aa_he = aa * HE
                    fbase = ix112 + (tv * (L * N * NF) + n * NF)
                    toff = tv * L
                    for hl in range(HG):
                        h = hg * HG + hl
                        if h < HE:
                            val = plsc.load_gather(ev, [aa_he + h])
                        else:
                            val = plsc.load_gather(fv, [fbase + (h - HE)])
                        ov[hl, n, pl.ds(toff, L)] = val

            pltpu.sync_copy(
                ov, out_hbm.at[b, pl.ds(hg * HG, HG), :, pl.ds(t0, TCH)])


def kernel(X, features, emb):
    x2 = X.astype(jnp.int32).reshape(B, T * N)
    f2 = features.reshape(B, T * N * NF)
    epad = jnp.zeros((NAA * HE + 19,), jnp.float32)
    epad = lax.dynamic_update_slice(epad, emb.reshape(-1), (0,))

    cp = pltpu.CompilerParams()
    if "needs_layout_passes" in pltpu.CompilerParams.__dataclass_fields__:
        cp = dataclasses.replace(cp, needs_layout_passes=False)
    mesh = plsc.VectorSubcoreMesh(core_axis_name="c", subcore_axis_name="s")
    k = pl.kernel(
        _sc_kernel,
        out_type=jax.ShapeDtypeStruct((B, H, N, T), jnp.float32),
        mesh=mesh,
        compiler_params=cp,
        scratch_types=[
            pltpu.VMEM((TCH * N,), jnp.int32),
            pltpu.VMEM((TCH * N * NF,), jnp.float32),
            pltpu.VMEM((NAA * HE + 19,), jnp.float32),
            pltpu.VMEM((HG, N, TCH), jnp.float32),
        ],
    )
    return k(x2, f2, epad)
